# baseline (speedup 1.0000x reference)
import jax
import jax.numpy as jnp
from jax import lax
from jax.experimental import pallas as pl
from jax.experimental.pallas import tpu as pltpu
from jax.experimental.pallas import tpu_sc as plsc

NC = 2
NS = 16
L = 16

ROWS = 128
COLS = 32768
ROWS_PER_W = ROWS // (NC * NS)
ZCHUNK = 8192
NZ = COLS // ZCHUNK
GROUP = 32
GELEMS = GROUP * L
QUARTER = COLS // 4
NGROUPS_Q = QUARTER // GELEMS


def _hardmax_body(logits_hbm, out_hbm, bufs, zero_buf, patch_buf, sem_z, sem_l):
    wid = lax.axis_index("s") * NC + lax.axis_index("c")
    base_row = wid * ROWS_PER_W
    lane_iota = lax.iota(jnp.int32, L)
    zeros16 = jnp.zeros((L,), jnp.float32)

    def zbody(g, _):
        base = g * (GROUP * L)
        for j in range(GROUP):
            zero_buf[pl.ds(pl.multiple_of(base + j * L, L), L)] = zeros16
        return 0

    lax.fori_loop(0, ZCHUNK // (GROUP * L), zbody, 0)

    def issue_row_loads(r):
        row = base_row + r
        return [
            pltpu.async_copy(
                logits_hbm.at[row, pl.ds(q * QUARTER, QUARTER)],
                bufs[(r % 2) * 4 + q],
                sem_l,
            )
            for q in range(4)
        ]

    loads = [issue_row_loads(0)]

    zcopies = []
    for r in range(ROWS_PER_W):
        row = base_row + r
        for k in range(NZ):
            zcopies.append(
                pltpu.async_copy(
                    zero_buf, out_hbm.at[row, pl.ds(k * ZCHUNK, ZCHUNK)], sem_z
                )
            )

    aligned_offsets = []
    for r in range(ROWS_PER_W):
        if r + 1 < ROWS_PER_W:
            loads.append(issue_row_loads(r + 1))
        buf_list = [bufs[(r % 2) * 4 + q] for q in range(4)]

        maxv = jnp.full((L,), -jnp.inf, jnp.float32)
        gidxv = jnp.zeros((L,), jnp.int32)
        carry = (maxv, gidxv, jnp.zeros((L,), jnp.int32))
        for q in range(4):
            loads[r][q].wait()
            buf = buf_list[q]

            def gbody(g, c, buf=buf):
                maxv, gidxv, gvec = c
                base = g * GELEMS
                ch = []
                for s in range(4):
                    acc = buf[pl.ds(pl.multiple_of(base + s * 8 * L, L), L)]
                    for j in range(1, 8):
                        acc = jnp.maximum(
                            acc,
                            buf[pl.ds(pl.multiple_of(base + (s * 8 + j) * L, L), L)],
                        )
                    ch.append(acc)
                gm = jnp.maximum(
                    jnp.maximum(ch[0], ch[1]), jnp.maximum(ch[2], ch[3])
                )
                m = gm > maxv
                maxv = jnp.maximum(maxv, gm)
                gidxv = jnp.where(m, gvec, gidxv)
                return maxv, gidxv, gvec + 1

            carry = lax.fori_loop(0, NGROUPS_Q, gbody, carry)
        maxv, gidxv, _ = carry

        gmax = jnp.max(maxv)
        bgroup = jnp.min(jnp.where(maxv == gmax, gidxv, jnp.int32(4 * NGROUPS_Q)))

        q_of = bgroup // NGROUPS_Q
        lgroup = bgroup - q_of * NGROUPS_Q
        gbase = lgroup * GELEMS
        best = jnp.full((L,), COLS, jnp.int32)
        for q in range(4):
            bq = jnp.full((L,), COLS, jnp.int32)
            for j in range(GROUP):
                off = pl.multiple_of(gbase + j * L, L)
                idx = q * QUARTER + gbase + j * L + lane_iota
                v = buf_list[q][pl.ds(off, L)]
                bq = jnp.minimum(bq, jnp.where(v == gmax, idx, jnp.int32(COLS)))
            best = jnp.where(q_of == q, bq, best)
        bidx = jnp.min(best)

        lane = lax.rem(bidx, jnp.int32(L))
        aligned_offsets.append(bidx - lane)
        patch_buf[pl.ds(r * L, L)] = jnp.where(
            lane_iota == lane, jnp.float32(1.0), jnp.float32(0.0)
        )

    for c in zcopies:
        c.wait()
    for r in range(ROWS_PER_W):
        row = base_row + r
        off = pl.multiple_of(aligned_offsets[r], L)
        pltpu.sync_copy(patch_buf.at[pl.ds(r * L, L)], out_hbm.at[row, pl.ds(off, L)])


@jax.jit
def kernel(logits):
    mesh = plsc.VectorSubcoreMesh(
        core_axis_name="c", subcore_axis_name="s", num_cores=NC, num_subcores=NS
    )
    run = pl.kernel(
        _hardmax_body,
        out_type=jax.ShapeDtypeStruct((ROWS, COLS), jnp.float32),
        mesh=mesh,
        scratch_types=[
            [pltpu.VMEM((QUARTER,), jnp.float32) for _ in range(8)],
            pltpu.VMEM((ZCHUNK,), jnp.float32),
            pltpu.VMEM((ROWS_PER_W * L,), jnp.float32),
            pltpu.SemaphoreType.DMA,
            pltpu.SemaphoreType.DMA,
        ],
        compiler_params=pltpu.CompilerParams(needs_layout_passes=False),
    )
    return run(logits)


# speedup vs baseline: 1.1221x; 1.1221x over previous
import functools

import jax
import jax.numpy as jnp
from jax import lax
from jax.experimental import pallas as pl
from jax.experimental.pallas import tpu as pltpu
from jax.experimental.pallas import tpu_sc as plsc

NC = 2
NS = 16
L = 16

ROWS = 128
COLS = 32768
ROWS_PER_W = ROWS // (NC * NS)
ZCHUNK = 8192
NZ = COLS // ZCHUNK
GROUP = 16
GELEMS = GROUP * L
NGROUPS = COLS // GELEMS


def _hardmax_body(
    logits_hbm, out_hbm, row_buf_a, row_buf_b, zero_buf, patch_buf, sem_z, sem_l
):
    wid = lax.axis_index("s") * NC + lax.axis_index("c")
    base_row = wid * ROWS_PER_W
    lane_iota = lax.iota(jnp.int32, L)
    zeros16 = jnp.zeros((L,), jnp.float32)

    def zbody(g, _):
        base = g * (GROUP * L)
        for j in range(GROUP):
            zero_buf[pl.ds(pl.multiple_of(base + j * L, L), L)] = zeros16
        return 0

    lax.fori_loop(0, ZCHUNK // (GROUP * L), zbody, 0)

    bufs = [row_buf_a, row_buf_b]

    load0 = pltpu.async_copy(logits_hbm.at[base_row], bufs[0], sem_l)

    zcopies = []
    for r in range(ROWS_PER_W):
        row = base_row + r
        for k in range(NZ):
            zcopies.append(
                pltpu.async_copy(
                    zero_buf, out_hbm.at[row, pl.ds(k * ZCHUNK, ZCHUNK)], sem_z
                )
            )

    loads = [load0]
    aligned_offsets = []
    for r in range(ROWS_PER_W):
        loads[r].wait()
        if r + 1 < ROWS_PER_W:
            loads.append(
                pltpu.async_copy(
                    logits_hbm.at[base_row + r + 1], bufs[(r + 1) % 2], sem_l
                )
            )
        row_buf = bufs[r % 2]

        def gbody(g, carry):
            maxv, gidxv, gvec = carry
            base = g * GELEMS
            ga = row_buf[pl.ds(pl.multiple_of(base, L), L)]
            gb = row_buf[pl.ds(pl.multiple_of(base + 8 * L, L), L)]
            for j in range(1, 8):
                ga = jnp.maximum(
                    ga, row_buf[pl.ds(pl.multiple_of(base + j * L, L), L)]
                )
                gb = jnp.maximum(
                    gb, row_buf[pl.ds(pl.multiple_of(base + (8 + j) * L, L), L)]
                )
            gm = jnp.maximum(ga, gb)
            m = gm > maxv
            maxv = jnp.maximum(maxv, gm)
            gidxv = jnp.where(m, gvec, gidxv)
            return maxv, gidxv, gvec + 1

        maxv0 = jnp.full((L,), -jnp.inf, jnp.float32)
        gidx0 = jnp.zeros((L,), jnp.int32)
        maxv, gidxv, _ = lax.fori_loop(
            0, NGROUPS, gbody, (maxv0, gidx0, jnp.zeros((L,), jnp.int32))
        )

        gmax = jnp.max(maxv)
        bgroup = jnp.min(jnp.where(maxv == gmax, gidxv, jnp.int32(NGROUPS)))

        gbase = bgroup * GELEMS
        best = jnp.full((L,), COLS, jnp.int32)
        for j in range(GROUP):
            off = pl.multiple_of(gbase + j * L, L)
            v = row_buf[pl.ds(off, L)]
            idx = gbase + j * L + lane_iota
            best = jnp.minimum(best, jnp.where(v == gmax, idx, jnp.int32(COLS)))
        bidx = jnp.min(best)

        lane = lax.rem(bidx, jnp.int32(L))
        aligned_offsets.append(bidx - lane)
        patch_buf[pl.ds(r * L, L)] = jnp.where(
            lane_iota == lane, jnp.float32(1.0), jnp.float32(0.0)
        )

    for c in zcopies:
        c.wait()
    for r in range(ROWS_PER_W):
        row = base_row + r
        off = pl.multiple_of(aligned_offsets[r], L)
        pltpu.sync_copy(patch_buf.at[pl.ds(r * L, L)], out_hbm.at[row, pl.ds(off, L)])


@jax.jit
def kernel(logits):
    mesh = plsc.VectorSubcoreMesh(
        core_axis_name="c", subcore_axis_name="s", num_cores=NC, num_subcores=NS
    )
    run = pl.kernel(
        _hardmax_body,
        out_type=jax.ShapeDtypeStruct((ROWS, COLS), jnp.float32),
        mesh=mesh,
        scratch_types=[
            pltpu.VMEM((COLS,), jnp.float32),
            pltpu.VMEM((COLS,), jnp.float32),
            pltpu.VMEM((ZCHUNK,), jnp.float32),
            pltpu.VMEM((ROWS_PER_W * L,), jnp.float32),
            pltpu.SemaphoreType.DMA,
            pltpu.SemaphoreType.DMA,
        ],
        compiler_params=pltpu.CompilerParams(
            needs_layout_passes=False, skip_device_barrier=True
        ),
    )
    return run(logits)


# speedup vs baseline: 1.1434x; 1.0189x over previous
import functools

import jax
import jax.numpy as jnp
from jax import lax
from jax.experimental import pallas as pl
from jax.experimental.pallas import tpu as pltpu
from jax.experimental.pallas import tpu_sc as plsc

NC = 2
NS = 16
L = 16

ROWS = 128
COLS = 32768
ROWS_PER_W = ROWS // (NC * NS)
ZCHUNK = 8192
NZ = COLS // ZCHUNK
GROUP = 16
GELEMS = GROUP * L
NGROUPS = COLS // GELEMS


def _hardmax_body(
    logits_hbm, out_hbm, row_buf_a, row_buf_b, zero_buf, patch_buf, sem_z, sem_l
):
    wid = lax.axis_index("s") * NC + lax.axis_index("c")
    base_row = wid * ROWS_PER_W
    lane_iota = lax.iota(jnp.int32, L)
    zeros16 = jnp.zeros((L,), jnp.float32)

    def zbody(g, _):
        base = g * (GROUP * L)
        for j in range(GROUP):
            zero_buf[pl.ds(pl.multiple_of(base + j * L, L), L)] = zeros16
        return 0

    lax.fori_loop(0, ZCHUNK // (GROUP * L), zbody, 0)

    bufs = [row_buf_a, row_buf_b]

    zcopies = []
    loads = []
    for r in range(ROWS_PER_W):
        row = base_row + r
        if r < 2:
            loads.append(pltpu.async_copy(logits_hbm.at[row], bufs[r % 2], sem_l))
        for k in range(NZ):
            zcopies.append(
                pltpu.async_copy(
                    zero_buf, out_hbm.at[row, pl.ds(k * ZCHUNK, ZCHUNK)], sem_z
                )
            )
    aligned_offsets = []
    for r in range(ROWS_PER_W):
        loads[r].wait()
        if r + 2 < ROWS_PER_W:
            loads.append(
                pltpu.async_copy(
                    logits_hbm.at[base_row + r + 2], bufs[(r + 2) % 2], sem_l
                )
            )
        row_buf = bufs[r % 2]

        def gbody(g, carry):
            maxv, gidxv, gvec = carry
            base = g * GELEMS
            ga = row_buf[pl.ds(pl.multiple_of(base, L), L)]
            gb = row_buf[pl.ds(pl.multiple_of(base + 8 * L, L), L)]
            for j in range(1, 8):
                ga = jnp.maximum(
                    ga, row_buf[pl.ds(pl.multiple_of(base + j * L, L), L)]
                )
                gb = jnp.maximum(
                    gb, row_buf[pl.ds(pl.multiple_of(base + (8 + j) * L, L), L)]
                )
            gm = jnp.maximum(ga, gb)
            m = gm > maxv
            maxv = jnp.maximum(maxv, gm)
            gidxv = jnp.where(m, gvec, gidxv)
            return maxv, gidxv, gvec + 1

        maxv0 = jnp.full((L,), -jnp.inf, jnp.float32)
        gidx0 = jnp.zeros((L,), jnp.int32)
        maxv, gidxv, _ = lax.fori_loop(
            0, NGROUPS, gbody, (maxv0, gidx0, jnp.zeros((L,), jnp.int32))
        )

        gmax = jnp.max(maxv)
        bgroup = jnp.min(jnp.where(maxv == gmax, gidxv, jnp.int32(NGROUPS)))

        gbase = bgroup * GELEMS
        best = jnp.full((L,), COLS, jnp.int32)
        for j in range(GROUP):
            off = pl.multiple_of(gbase + j * L, L)
            v = row_buf[pl.ds(off, L)]
            idx = gbase + j * L + lane_iota
            best = jnp.minimum(best, jnp.where(v == gmax, idx, jnp.int32(COLS)))
        bidx = jnp.min(best)

        lane = lax.rem(bidx, jnp.int32(L))
        aligned_offsets.append(bidx - lane)
        patch_buf[pl.ds(r * L, L)] = jnp.where(
            lane_iota == lane, jnp.float32(1.0), jnp.float32(0.0)
        )

    for c in zcopies:
        c.wait()
    for r in range(ROWS_PER_W):
        row = base_row + r
        off = pl.multiple_of(aligned_offsets[r], L)
        pltpu.sync_copy(patch_buf.at[pl.ds(r * L, L)], out_hbm.at[row, pl.ds(off, L)])


@jax.jit
def kernel(logits):
    mesh = plsc.VectorSubcoreMesh(
        core_axis_name="c", subcore_axis_name="s", num_cores=NC, num_subcores=NS
    )
    run = pl.kernel(
        _hardmax_body,
        out_type=jax.ShapeDtypeStruct((ROWS, COLS), jnp.float32),
        mesh=mesh,
        scratch_types=[
            pltpu.VMEM((COLS,), jnp.float32),
            pltpu.VMEM((COLS,), jnp.float32),
            pltpu.VMEM((ZCHUNK,), jnp.float32),
            pltpu.VMEM((ROWS_PER_W * L,), jnp.float32),
            pltpu.SemaphoreType.DMA,
            pltpu.SemaphoreType.DMA,
        ],
        compiler_params=pltpu.CompilerParams(
            needs_layout_passes=False, skip_device_barrier=True
        ),
    )
    return run(logits)
